# Initial kernel scaffold; baseline (speedup 1.0000x reference)
import jax
import jax.numpy as jnp
from jax.experimental import pallas as pl


def kernel(input, weight_values, bias_values, weight_indices, bias_indices):
    raise NotImplementedError("write your pallas kernel here")


# speedup vs baseline: 3.0658x; 3.0658x over previous
import functools

import jax
import jax.numpy as jnp
from jax import lax
from jax.experimental import pallas as pl
from jax.experimental.pallas import tpu as pltpu
from jax.experimental.pallas import tpu_sc as plsc

NC = 2
NS = 16
L = 16
K = 16


def _sc_spmm(n_rows_x, n_out, bh, per_tile):
    n_chunks = per_tile // K
    rps = n_out // NS

    mesh = plsc.VectorSubcoreMesh(
        core_axis_name="c", subcore_axis_name="s",
        num_cores=NC, num_subcores=NS)

    @functools.partial(
        pl.kernel,
        out_type=jax.ShapeDtypeStruct((NC * n_out, bh), jnp.float32),
        mesh=mesh,
        scratch_types=[
            pltpu.VMEM((per_tile,), jnp.int32),
            pltpu.VMEM((per_tile,), jnp.int32),
            pltpu.VMEM((per_tile,), jnp.float32),
            pltpu.VMEM((K, bh), jnp.float32),
            pltpu.VMEM_SHARED((n_out, bh), jnp.float32),
            pltpu.SemaphoreType.DMA,
        ],
    )
    def body(x_hbm, cols_hbm, rows_hbm, vals_hbm, out_hbm,
             cols_v, rows_v, vals_v, gbuf, acc_sh, sem):
        c = lax.axis_index("c")
        s = lax.axis_index("s")
        base = pl.multiple_of(s * per_tile, 8)

        pltpu.sync_copy(cols_hbm.at[pl.ds(base, per_tile)], cols_v)
        pltpu.sync_copy(rows_hbm.at[pl.ds(base, per_tile)], rows_v)
        pltpu.sync_copy(vals_hbm.at[pl.ds(base, per_tile)], vals_v)

        zero = jnp.zeros((L,), jnp.float32)
        for i in range(K):
            for j in range(bh // L):
                gbuf[i, pl.ds(j * L, L)] = zero
        for i in range(rps // K):
            pltpu.sync_copy(gbuf, acc_sh.at[pl.ds(s * rps + i * K, K)])
        plsc.subcore_barrier()

        x_row_base = c * n_rows_x

        def chunk(g, carry):
            off = pl.multiple_of(g * K, 8)
            cidx = cols_v[pl.ds(off, K)] + x_row_base
            pltpu.async_copy(x_hbm.at[cidx], gbuf, sem).wait()
            w16 = vals_v[pl.ds(off, K)]
            for k in range(K):
                wb = lax.gather(
                    w16, jnp.full((L, 1), k, jnp.int32),
                    lax.GatherDimensionNumbers(
                        offset_dims=(), collapsed_slice_dims=(0,),
                        start_index_map=(0,)),
                    (1,), mode=lax.GatherScatterMode.PROMISE_IN_BOUNDS)
                for j in range(bh // L):
                    sl = pl.ds(j * L, L)
                    gbuf[k, sl] = gbuf[k, sl] * wb
            ridx = rows_v[pl.ds(off, K)]
            pltpu.sync_copy(gbuf, acc_sh.at[ridx], add=True)
            return carry

        lax.fori_loop(0, n_chunks, chunk, 0)
        plsc.subcore_barrier()

        dst_base = c * n_out + s * rps
        pltpu.sync_copy(acc_sh.at[pl.ds(s * rps, rps)],
                        out_hbm.at[pl.ds(dst_base, rps)])

    return body


def kernel(input, weight_values, bias_values, weight_indices, bias_indices):
    b, n_in = input.shape
    n_out = n_in
    bh = b // NC
    nnz = weight_values.shape[0]
    bnnz = bias_values.shape[0]

    tot = nnz + bnnz
    per_tile = -(-tot // (NS * K)) * K
    pad = NS * per_tile - tot
    cols = jnp.concatenate([
        weight_indices[1],
        jnp.full((bnnz,), n_in, jnp.int32),
        jnp.zeros((pad,), jnp.int32),
    ])
    rows = jnp.concatenate([
        weight_indices[0], bias_indices, jnp.zeros((pad,), jnp.int32)])
    vals = jnp.concatenate([
        weight_values, bias_values, jnp.zeros((pad,), jnp.float32)])

    xt = jnp.concatenate([input, jnp.ones((b, 1), input.dtype)], axis=1).T
    x_flat = xt.reshape(n_in + 1, NC, bh).transpose(1, 0, 2)
    x_flat = x_flat.reshape(NC * (n_in + 1), bh)

    out_flat = _sc_spmm(n_in + 1, n_out, bh, per_tile)(x_flat, cols, rows, vals)

    out_t = out_flat.reshape(NC, n_out, bh)
    return jnp.concatenate([out_t[0].T, out_t[1].T], axis=0)


# speedup vs baseline: 6.0747x; 1.9814x over previous
import functools

import jax
import jax.numpy as jnp
from jax import lax
from jax.experimental import pallas as pl
from jax.experimental.pallas import tpu as pltpu
from jax.experimental.pallas import tpu_sc as plsc

NC = 2
NS = 16
L = 16
K = 16
NBUF = 3


def _sc_spmm(n_rows_x, n_out, bh, per_tile):
    n_chunks = per_tile // K
    rps = n_out // NS

    mesh = plsc.VectorSubcoreMesh(
        core_axis_name="c", subcore_axis_name="s",
        num_cores=NC, num_subcores=NS)

    @functools.partial(
        pl.kernel,
        out_type=jax.ShapeDtypeStruct((NC * n_out, bh), jnp.float32),
        mesh=mesh,
        scratch_types=[
            pltpu.VMEM((per_tile,), jnp.int32),
            pltpu.VMEM((per_tile,), jnp.int32),
            pltpu.VMEM((per_tile,), jnp.float32),
            pltpu.VMEM((NBUF, K, bh), jnp.float32),
            pltpu.VMEM_SHARED((n_out, bh), jnp.float32),
            tuple(pltpu.SemaphoreType.DMA for _ in range(NBUF)),
            tuple(pltpu.SemaphoreType.DMA for _ in range(NBUF)),
        ],
    )
    def body(x_hbm, cols_hbm, rows_hbm, vals_hbm, out_hbm,
             cols_v, rows_v, vals_v, gbuf, acc_sh, gsem, ssem):
        c = lax.axis_index("c")
        s = lax.axis_index("s")
        base = pl.multiple_of(s * per_tile, 8)

        pltpu.sync_copy(cols_hbm.at[pl.ds(base, per_tile)], cols_v)
        pltpu.sync_copy(rows_hbm.at[pl.ds(base, per_tile)], rows_v)
        pltpu.sync_copy(vals_hbm.at[pl.ds(base, per_tile)], vals_v)

        zero = jnp.zeros((L,), jnp.float32)
        for i in range(K):
            for j in range(bh // L):
                gbuf[0, i, pl.ds(j * L, L)] = zero
        for i in range(rps // K):
            pltpu.sync_copy(gbuf.at[0], acc_sh.at[pl.ds(s * rps + i * K, K)])
        plsc.subcore_barrier()

        x_row_base = c * n_rows_x

        def start_gather(g, buf):
            off = pl.multiple_of(g * K, 8)
            cidx = cols_v[pl.ds(off, K)] + x_row_base
            pltpu.async_copy(x_hbm.at[cidx], gbuf.at[buf], gsem[buf])

        def wait_gather(buf):
            pltpu.make_async_copy(x_hbm.at[pl.ds(0, K)], gbuf.at[buf],
                                  gsem[buf]).wait()

        def wait_scatter(buf):
            pltpu.make_async_copy(gbuf.at[buf], acc_sh.at[pl.ds(0, K)],
                                  ssem[buf]).wait()

        assert n_chunks % NBUF == 0
        start_gather(0, 0)

        @pl.loop(0, n_chunks // NBUF)
        def pair(p):
            for u in range(NBUF):
                b = u
                nb = (u + 1) % NBUF
                g = p * NBUF + u

                if u == NBUF - 1:
                    wait_scatter(nb)
                else:
                    @pl.when(p >= 1)
                    def _():
                        wait_scatter(nb)

                if u == NBUF - 1:
                    @pl.when(p + 1 < n_chunks // NBUF)
                    def _():
                        start_gather(g + 1, nb)
                else:
                    start_gather(g + 1, nb)

                wait_gather(b)
                off = pl.multiple_of(g * K, 8)
                w16 = vals_v[pl.ds(off, K)]
                for k in range(K):
                    wb = lax.gather(
                        w16, jnp.full((L, 1), k, jnp.int32),
                        lax.GatherDimensionNumbers(
                            offset_dims=(), collapsed_slice_dims=(0,),
                            start_index_map=(0,)),
                        (1,), mode=lax.GatherScatterMode.PROMISE_IN_BOUNDS)
                    for j in range(bh // L):
                        sl = pl.ds(j * L, L)
                        gbuf[b, k, sl] = gbuf[b, k, sl] * wb
                ridx = rows_v[pl.ds(off, K)]
                pltpu.async_copy(gbuf.at[b], acc_sh.at[ridx], ssem[b],
                                 add=True)

        for g in range(n_chunks - NBUF + 1, n_chunks):
            wait_scatter(g % NBUF)
        plsc.subcore_barrier()

        dst_base = c * n_out + s * rps
        pltpu.sync_copy(acc_sh.at[pl.ds(s * rps, rps)],
                        out_hbm.at[pl.ds(dst_base, rps)])

    return body


def kernel(input, weight_values, bias_values, weight_indices, bias_indices):
    b, n_in = input.shape
    n_out = n_in
    bh = b // NC
    nnz = weight_values.shape[0]
    bnnz = bias_values.shape[0]

    tot = nnz + bnnz
    per_tile = -(-tot // (NS * K * NBUF)) * (K * NBUF)
    pad = NS * per_tile - tot
    cols = jnp.concatenate([
        weight_indices[1],
        jnp.full((bnnz,), n_in, jnp.int32),
        jnp.zeros((pad,), jnp.int32),
    ])
    rows = jnp.concatenate([
        weight_indices[0], bias_indices, jnp.zeros((pad,), jnp.int32)])
    vals = jnp.concatenate([
        weight_values, bias_values, jnp.zeros((pad,), jnp.float32)])

    xt = jnp.concatenate([input, jnp.ones((b, 1), input.dtype)], axis=1).T
    x_flat = xt.reshape(n_in + 1, NC, bh).transpose(1, 0, 2)
    x_flat = x_flat.reshape(NC * (n_in + 1), bh)

    out_flat = _sc_spmm(n_in + 1, n_out, bh, per_tile)(x_flat, cols, rows, vals)

    out_t = out_flat.reshape(NC, n_out, bh)
    return jnp.concatenate([out_t[0].T, out_t[1].T], axis=0)
